# baseline (speedup 1.0000x reference)
import functools

import jax
import jax.numpy as jnp
from jax import lax
from jax.experimental import pallas as pl
from jax.experimental.pallas import tpu as pltpu
from jax.experimental.pallas import tpu_sc as plsc

N_EXPERTS = 8
TOP_K = 2
LOAD_FACTOR = 1.25
NC, NS, NLANE = 2, 16, 16
NW = NC * NS


def _route_body(x_ref, wr_ref, p01_ref, g01_ref, s01_ref, *, B, T, cap):
    E = N_EXPERTS
    nrows = E * B * cap
    logits = jnp.dot(x_ref[...], wr_ref[...],
                     preferred_element_type=jnp.float32)
    m = jnp.max(logits, axis=-1, keepdims=True)
    ex = jnp.exp(logits - m)
    probs = ex / jnp.sum(ex, axis=-1, keepdims=True)

    lane = jax.lax.broadcasted_iota(jnp.int32, probs.shape, 1)
    m0 = jnp.max(probs, axis=-1, keepdims=True)
    e0 = jnp.min(jnp.where(probs == m0, lane, E), axis=-1, keepdims=True)
    pm = jnp.where(lane == e0, -jnp.inf, probs)
    m1 = jnp.max(pm, axis=-1, keepdims=True)
    e1 = jnp.min(jnp.where(pm == m1, lane, E), axis=-1, keepdims=True)

    oh0 = (lane == e0).astype(jnp.float32)
    oh1 = (lane == e1).astype(jnp.float32)

    S = 512
    nblk = T // S
    r = jax.lax.broadcasted_iota(jnp.int32, (S, S), 0)
    c = jax.lax.broadcasted_iota(jnp.int32, (S, S), 1)
    stril = (r > c).astype(jnp.float32)

    for b in range(B):
        sl = slice(b * T, (b + 1) * T)
        o0 = oh0[sl]
        o1 = oh1[sl]
        carry = jnp.zeros((1, E), jnp.float32)
        pos = []
        for o in (o0, o1):
            pieces = []
            for blk in range(nblk):
                ob = o[blk * S:(blk + 1) * S]
                cb = jnp.dot(stril, ob, preferred_element_type=jnp.float32)
                pieces.append(jnp.sum((cb + carry) * ob, axis=-1))
                carry = carry + jnp.sum(ob, axis=0, keepdims=True)
            pos.append(jnp.concatenate(pieces, axis=0))
        pos0i = pos[0].astype(jnp.int32)
        pos1i = pos[1].astype(jnp.int32)
        e0b = e0[sl, 0]
        e1b = e1[sl, 0]

        g0 = e0b * (B * cap) + b * cap + jnp.minimum(pos0i, cap - 1)
        g1 = e1b * (B * cap) + b * cap + jnp.minimum(pos1i, cap - 1)
        g01_ref[0, sl] = g0
        g01_ref[1, sl] = g1
        s0 = e0b * (B * cap) + b * cap + pos0i
        s1 = e1b * (B * cap) + b * cap + pos1i
        s01_ref[0, sl] = jnp.where(pos0i < cap, s0, nrows)
        s01_ref[1, sl] = jnp.where(pos1i < cap, s1, nrows)
        p01_ref[0, sl, :] = jnp.broadcast_to(m0[sl], (T, 16))
        p01_ref[1, sl, :] = jnp.broadcast_to(m1[sl], (T, 16))


def _sc_scatter_body(x_hbm, s_hbm, out_hbm, idx0_v, idx1_v, rows0_v,
                     rows1_v, semr0, semr1, semw0, semw1, *,
                     asg_per_w, chunk, BT):
    wid = lax.axis_index("s") * NC + lax.axis_index("c")
    n = asg_per_w // chunk
    a_base = wid * asg_per_w
    bt_base = lax.rem(a_base, BT)
    idx = (idx0_v, idx1_v)
    rows = (rows0_v, rows1_v)
    semr = (semr0, semr1)
    semw = (semw0, semw1)
    reads = [None, None]
    wrb = [None, None]

    def fire(ch):
        p = ch % 2
        pltpu.sync_copy(s_hbm.at[pl.ds(a_base + ch * chunk, chunk)], idx[p])
        reads[p] = pltpu.async_copy(
            x_hbm.at[pl.ds(bt_base + ch * chunk, chunk)], rows[p], semr[p])

    fire(0)
    for ch in range(n):
        p = ch % 2
        if ch + 1 < n:
            if wrb[1 - p] is not None:
                wrb[1 - p].wait()
            fire(ch + 1)
        reads[p].wait()
        wrb[p] = pltpu.async_copy(rows[p], out_hbm.at[idx[p]], semw[p])
    for p in range(2):
        if wrb[p] is not None:
            wrb[p].wait()


def _moe_body(xe_ref, w1_ref, w2_ref, w3_ref, p01_ref, g01_ref,
              y_ref, oacc_ref, *, B, T, cap, n_h):
    e = pl.program_id(0)
    h = pl.program_id(1)

    xe = xe_ref[...]
    hh = jnp.dot(xe, w1_ref[0], preferred_element_type=jnp.float32)
    gg = jnp.dot(xe, w2_ref[0], preferred_element_type=jnp.float32)
    gg = gg * (1.0 / (1.0 + jnp.exp(-gg)))
    act = gg * hh
    o_tile = jnp.dot(act, w3_ref[0], preferred_element_type=jnp.float32)

    @pl.when(h == 0)
    def _init_o():
        oacc_ref[...] = o_tile

    @pl.when(h != 0)
    def _acc_o():
        oacc_ref[...] = oacc_ref[...] + o_tile

    @pl.when(h == n_h - 1)
    def _combine():
        ciota = jax.lax.broadcasted_iota(jnp.int32, (T, cap), 1)
        for b in range(B):
            sl = slice(b * T, (b + 1) * T)
            base_row = e * (B * cap) + b * cap
            csum = None
            for k in range(TOP_K):
                gk = g01_ref[k, sl][:, None]
                pk = p01_ref[k, sl, 0][:, None]
                ck = jnp.where(gk == base_row + ciota, pk, 0.0)
                csum = ck if csum is None else csum + ck
            contrib = jnp.dot(csum, oacc_ref[b * cap:(b + 1) * cap, :],
                              preferred_element_type=jnp.float32)

            @pl.when(e == 0)
            def _init_y():
                y_ref[sl, :] = contrib

            @pl.when(e != 0)
            def _acc_y():
                y_ref[sl, :] = y_ref[sl, :] + contrib


def kernel(x, W_r, w_c_fc, w_gate, w_c_proj):
    B, T, C = x.shape
    E, _, H = w_c_fc.shape
    cap = int(LOAD_FACTOR * TOP_K * max(1, T / E))
    NROWS = E * B * cap
    xf = x.reshape(B * T, C)

    p01, g01, s01 = pl.pallas_call(
        functools.partial(_route_body, B=B, T=T, cap=cap),
        out_shape=(
            jax.ShapeDtypeStruct((TOP_K, B * T, 16), jnp.float32),
            jax.ShapeDtypeStruct((TOP_K, B * T), jnp.int32),
            jax.ShapeDtypeStruct((TOP_K, B * T), jnp.int32),
        ),
    )(xf, W_r)
    s_flat = s01.reshape(TOP_K * B * T)

    mesh = plsc.VectorSubcoreMesh(core_axis_name="c", subcore_axis_name="s")
    asg_per_w = (TOP_K * B * T) // NW
    dchunk = 64
    xep = pl.kernel(
        functools.partial(_sc_scatter_body, asg_per_w=asg_per_w,
                          chunk=dchunk, BT=B * T),
        out_type=jax.ShapeDtypeStruct((NROWS + 8, C), jnp.float32),
        mesh=mesh,
        scratch_types=[
            pltpu.VMEM((dchunk,), jnp.int32),
            pltpu.VMEM((dchunk,), jnp.int32),
            pltpu.VMEM((dchunk, C), jnp.float32),
            pltpu.VMEM((dchunk, C), jnp.float32),
            pltpu.SemaphoreType.DMA,
            pltpu.SemaphoreType.DMA,
            pltpu.SemaphoreType.DMA,
            pltpu.SemaphoreType.DMA,
        ],
    )(xf, s_flat)

    HT = 512
    n_h = H // HT
    y = pl.pallas_call(
        functools.partial(_moe_body, B=B, T=T, cap=cap, n_h=n_h),
        grid=(E, n_h),
        in_specs=[
            pl.BlockSpec((B * cap, C), lambda e, h: (e, 0)),
            pl.BlockSpec((1, C, HT), lambda e, h: (e, 0, h)),
            pl.BlockSpec((1, C, HT), lambda e, h: (e, 0, h)),
            pl.BlockSpec((1, HT, C), lambda e, h: (e, h, 0)),
            pl.BlockSpec((TOP_K, B * T, 16), lambda e, h: (0, 0, 0)),
            pl.BlockSpec((TOP_K, B * T), lambda e, h: (0, 0)),
        ],
        out_specs=pl.BlockSpec((B * T, C), lambda e, h: (0, 0)),
        out_shape=jax.ShapeDtypeStruct((B * T, C), jnp.float32),
        scratch_shapes=[pltpu.VMEM((B * cap, C), jnp.float32)],
        compiler_params=pltpu.CompilerParams(
            vmem_limit_bytes=100 * 1024 * 1024,
        ),
    )(xep, w_c_fc, w_gate, w_c_proj, p01, g01)

    return y.reshape(B, T, C)


# speedup vs baseline: 1.2216x; 1.2216x over previous
import functools

import jax
import jax.numpy as jnp
from jax import lax
from jax.experimental import pallas as pl
from jax.experimental.pallas import tpu as pltpu
from jax.experimental.pallas import tpu_sc as plsc

N_EXPERTS = 8
TOP_K = 2
LOAD_FACTOR = 1.25
NC, NS, NLANE = 2, 16, 16
NW = NC * NS


def _route_body(x_ref, wr_ref, p01_ref, g01_ref, s01_ref, *, B, T, cap):
    E = N_EXPERTS
    nrows = E * B * cap
    logits = jnp.dot(x_ref[...], wr_ref[...],
                     preferred_element_type=jnp.float32)
    m = jnp.max(logits, axis=-1, keepdims=True)
    ex = jnp.exp(logits - m)
    probs = ex / jnp.sum(ex, axis=-1, keepdims=True)

    lane = jax.lax.broadcasted_iota(jnp.int32, probs.shape, 1)
    m0 = jnp.max(probs, axis=-1, keepdims=True)
    e0 = jnp.min(jnp.where(probs == m0, lane, E), axis=-1, keepdims=True)
    pm = jnp.where(lane == e0, -jnp.inf, probs)
    m1 = jnp.max(pm, axis=-1, keepdims=True)
    e1 = jnp.min(jnp.where(pm == m1, lane, E), axis=-1, keepdims=True)

    oh0 = (lane == e0).astype(jnp.float32)
    oh1 = (lane == e1).astype(jnp.float32)

    S = 512
    nblk = T // S
    r = jax.lax.broadcasted_iota(jnp.int32, (S, S), 0)
    c = jax.lax.broadcasted_iota(jnp.int32, (S, S), 1)
    stril = (r > c).astype(jnp.float32)

    for b in range(B):
        sl = slice(b * T, (b + 1) * T)
        o0 = oh0[sl]
        o1 = oh1[sl]
        carry = jnp.zeros((1, E), jnp.float32)
        pos = []
        for o in (o0, o1):
            pieces = []
            for blk in range(nblk):
                ob = o[blk * S:(blk + 1) * S]
                cb = jnp.dot(stril, ob, preferred_element_type=jnp.float32)
                pieces.append(jnp.sum((cb + carry) * ob, axis=-1))
                carry = carry + jnp.sum(ob, axis=0, keepdims=True)
            pos.append(jnp.concatenate(pieces, axis=0))
        pos0i = pos[0].astype(jnp.int32)
        pos1i = pos[1].astype(jnp.int32)
        e0b = e0[sl, 0]
        e1b = e1[sl, 0]

        g0 = e0b * (B * cap) + b * cap + jnp.minimum(pos0i, cap - 1)
        g1 = e1b * (B * cap) + b * cap + jnp.minimum(pos1i, cap - 1)
        g01_ref[0, sl] = g0
        g01_ref[1, sl] = g1
        s0 = e0b * (B * cap) + b * cap + pos0i
        s1 = e1b * (B * cap) + b * cap + pos1i
        s01_ref[0, sl] = jnp.where(pos0i < cap, s0, nrows)
        s01_ref[1, sl] = jnp.where(pos1i < cap, s1, nrows)
        p01_ref[0, sl, :] = jnp.broadcast_to(m0[sl], (T, 16))
        p01_ref[1, sl, :] = jnp.broadcast_to(m1[sl], (T, 16))


def _expert_body(xe_ref, w1_ref, w2_ref, w3_ref, eo_ref, oacc_ref, *, n_h):
    h = pl.program_id(1)
    xe = xe_ref[...].astype(jnp.bfloat16)
    hh = jnp.dot(xe, w1_ref[0].astype(jnp.bfloat16),
                 preferred_element_type=jnp.float32)
    gg = jnp.dot(xe, w2_ref[0].astype(jnp.bfloat16),
                 preferred_element_type=jnp.float32)
    gg = gg * (1.0 / (1.0 + jnp.exp(-gg)))
    act = gg * hh
    o_tile = jnp.dot(act.astype(jnp.bfloat16),
                     w3_ref[0].astype(jnp.bfloat16),
                     preferred_element_type=jnp.float32)

    @pl.when(h == 0)
    def _init():
        oacc_ref[...] = o_tile

    @pl.when(h != 0)
    def _acc():
        oacc_ref[...] = oacc_ref[...] + o_tile

    @pl.when(h == n_h - 1)
    def _out():
        eo_ref[...] = oacc_ref[...]


def _sc_scatter_body(x_hbm, s_hbm, out_hbm, idx0_v, idx1_v, rows0_v,
                     rows1_v, semr0, semr1, semw0, semw1, *,
                     asg_per_w, chunk, BT):
    wid = lax.axis_index("s") * NC + lax.axis_index("c")
    n = asg_per_w // chunk
    a_base = wid * asg_per_w
    bt_base = lax.rem(a_base, BT)
    idx = (idx0_v, idx1_v)
    rows = (rows0_v, rows1_v)
    semr = (semr0, semr1)
    semw = (semw0, semw1)
    reads = [None, None]
    wrb = [None, None]

    def fire(ch):
        p = ch % 2
        pltpu.sync_copy(s_hbm.at[pl.ds(a_base + ch * chunk, chunk)], idx[p])
        reads[p] = pltpu.async_copy(
            x_hbm.at[pl.ds(bt_base + ch * chunk, chunk)], rows[p], semr[p])

    fire(0)
    for ch in range(n):
        p = ch % 2
        if ch + 1 < n:
            if wrb[1 - p] is not None:
                wrb[1 - p].wait()
            fire(ch + 1)
        reads[p].wait()
        wrb[p] = pltpu.async_copy(rows[p], out_hbm.at[idx[p]], semw[p])
    for p in range(2):
        if wrb[p] is not None:
            wrb[p].wait()


def _sc_combine_body(eo_hbm, g0_hbm, g1_hbm, p0_hbm, p1_hbm, y_hbm,
                     g0a, g1a, p0a, p1a, r0a, r1a,
                     g0b, g1b, p0b, p1b, r0b, r1b,
                     sga0, sga1, sgb0, sgb1, swa, swb, *,
                     tok_per_w, chunk, C):
    wid = lax.axis_index("s") * NC + lax.axis_index("c")
    nv = C // NLANE
    n = tok_per_w // chunk
    bufs = ((g0a, g1a, p0a, p1a, r0a, r1a, sga0, sga1, swa),
            (g0b, g1b, p0b, p1b, r0b, r1b, sgb0, sgb1, swb))
    gath = [None, None]
    wrb = [None, None]

    def fire(ch):
        p = ch % 2
        G0, G1, P0, P1, R0, R1, SG0, SG1, _ = bufs[p]
        base = wid * tok_per_w + ch * chunk
        pltpu.sync_copy(g0_hbm.at[pl.ds(base, chunk)], G0)
        pltpu.sync_copy(g1_hbm.at[pl.ds(base, chunk)], G1)
        pltpu.sync_copy(p0_hbm.at[pl.ds(base, chunk)], P0)
        pltpu.sync_copy(p1_hbm.at[pl.ds(base, chunk)], P1)
        gath[p] = (pltpu.async_copy(eo_hbm.at[G0], R0, SG0),
                   pltpu.async_copy(eo_hbm.at[G1], R1, SG1))

    fire(0)
    for ch in range(n):
        p = ch % 2
        _, _, P0, P1, R0, R1, _, _, SW = bufs[p]
        if ch + 1 < n:
            if wrb[1 - p] is not None:
                wrb[1 - p].wait()
            fire(ch + 1)
        gath[p][0].wait()
        gath[p][1].wait()

        def row_body(rr, carry):
            a0 = P0[rr, :]
            a1 = P1[rr, :]
            for v in range(nv):
                x0 = R0[rr, pl.ds(v * NLANE, NLANE)]
                x1 = R1[rr, pl.ds(v * NLANE, NLANE)]
                R0[rr, pl.ds(v * NLANE, NLANE)] = a0 * x0 + a1 * x1
            return carry

        lax.fori_loop(0, chunk, row_body, 0)
        base = wid * tok_per_w + ch * chunk
        wrb[p] = pltpu.async_copy(R0, y_hbm.at[pl.ds(base, chunk)], SW)
    for p in range(2):
        if wrb[p] is not None:
            wrb[p].wait()


def kernel(x, W_r, w_c_fc, w_gate, w_c_proj):
    B, T, C = x.shape
    E, _, H = w_c_fc.shape
    cap = int(LOAD_FACTOR * TOP_K * max(1, T / E))
    NROWS = E * B * cap
    xf = x.reshape(B * T, C)

    p01, g01, s01 = pl.pallas_call(
        functools.partial(_route_body, B=B, T=T, cap=cap),
        out_shape=(
            jax.ShapeDtypeStruct((TOP_K, B * T, 16), jnp.float32),
            jax.ShapeDtypeStruct((TOP_K, B * T), jnp.int32),
            jax.ShapeDtypeStruct((TOP_K, B * T), jnp.int32),
        ),
    )(xf, W_r)
    s_flat = s01.reshape(TOP_K * B * T)

    mesh = plsc.VectorSubcoreMesh(core_axis_name="c", subcore_axis_name="s")
    asg_per_w = (TOP_K * B * T) // NW
    dchunk = 64
    xep = pl.kernel(
        functools.partial(_sc_scatter_body, asg_per_w=asg_per_w,
                          chunk=dchunk, BT=B * T),
        out_type=jax.ShapeDtypeStruct((NROWS + 8, C), jnp.float32),
        mesh=mesh,
        scratch_types=[
            pltpu.VMEM((dchunk,), jnp.int32),
            pltpu.VMEM((dchunk,), jnp.int32),
            pltpu.VMEM((dchunk, C), jnp.float32),
            pltpu.VMEM((dchunk, C), jnp.float32),
            pltpu.SemaphoreType.DMA,
            pltpu.SemaphoreType.DMA,
            pltpu.SemaphoreType.DMA,
            pltpu.SemaphoreType.DMA,
        ],
    )(xf, s_flat)

    HT = 768
    n_h = H // HT
    eo = pl.pallas_call(
        functools.partial(_expert_body, n_h=n_h),
        grid=(E, n_h),
        in_specs=[
            pl.BlockSpec((B * cap, C), lambda e, h: (e, 0)),
            pl.BlockSpec((1, C, HT), lambda e, h: (e, 0, h)),
            pl.BlockSpec((1, C, HT), lambda e, h: (e, 0, h)),
            pl.BlockSpec((1, HT, C), lambda e, h: (e, h, 0)),
        ],
        out_specs=pl.BlockSpec((B * cap, C), lambda e, h: (e, 0)),
        out_shape=jax.ShapeDtypeStruct((NROWS, C), jnp.float32),
        scratch_shapes=[pltpu.VMEM((B * cap, C), jnp.float32)],
        compiler_params=pltpu.CompilerParams(
            vmem_limit_bytes=100 * 1024 * 1024,
        ),
    )(xep, w_c_fc, w_gate, w_c_proj)

    tok_per_w = (B * T) // NW
    cchunk = 32
    y = pl.kernel(
        functools.partial(_sc_combine_body, tok_per_w=tok_per_w,
                          chunk=cchunk, C=C),
        out_type=jax.ShapeDtypeStruct((B * T, C), jnp.float32),
        mesh=mesh,
        scratch_types=(
            [pltpu.VMEM((cchunk,), jnp.int32)] * 2
            + [pltpu.VMEM((cchunk, 16), jnp.float32)] * 2
            + [pltpu.VMEM((cchunk, C), jnp.float32)] * 2
            + [pltpu.VMEM((cchunk,), jnp.int32)] * 2
            + [pltpu.VMEM((cchunk, 16), jnp.float32)] * 2
            + [pltpu.VMEM((cchunk, C), jnp.float32)] * 2
            + [pltpu.SemaphoreType.DMA] * 6
        ),
    )(eo, g01[0], g01[1], p01[0], p01[1])

    return y.reshape(B, T, C)


# speedup vs baseline: 1.2225x; 1.0007x over previous
import functools

import jax
import jax.numpy as jnp
from jax import lax
from jax.experimental import pallas as pl
from jax.experimental.pallas import tpu as pltpu
from jax.experimental.pallas import tpu_sc as plsc

N_EXPERTS = 8
TOP_K = 2
LOAD_FACTOR = 1.25
NC, NS, NLANE = 2, 16, 16
NW = NC * NS


def _route_body(x_ref, wr_ref, p01_ref, g01_ref, s01_ref, *, B, T, cap):
    E = N_EXPERTS
    nrows = E * B * cap
    logits = jnp.dot(x_ref[...], wr_ref[...],
                     preferred_element_type=jnp.float32)
    lgt = logits.T

    m = jnp.max(lgt, axis=0, keepdims=True)
    ex = jnp.exp(lgt - m)
    probs = ex / jnp.sum(ex, axis=0, keepdims=True)

    srow = jax.lax.broadcasted_iota(jnp.int32, probs.shape, 0)
    m0 = jnp.max(probs, axis=0, keepdims=True)
    e0 = jnp.min(jnp.where(probs == m0, srow, E), axis=0, keepdims=True)
    pm = jnp.where(srow == e0, -jnp.inf, probs)
    m1 = jnp.max(pm, axis=0, keepdims=True)
    e1 = jnp.min(jnp.where(pm == m1, srow, E), axis=0, keepdims=True)

    oh0 = (srow == e0).astype(jnp.float32)
    oh1 = (srow == e1).astype(jnp.float32)

    r = jax.lax.broadcasted_iota(jnp.int32, (T, T), 0)
    c = jax.lax.broadcasted_iota(jnp.int32, (T, T), 1)
    U = (r < c).astype(jnp.float32)

    for b in range(B):
        sl = slice(b * T, (b + 1) * T)
        o0 = oh0[:, sl]
        o1 = oh1[:, sl]
        c0 = jnp.dot(o0, U, preferred_element_type=jnp.float32)
        tot0 = c0[:, T - 1:T] + o0[:, T - 1:T]
        c1 = jnp.dot(o1, U, preferred_element_type=jnp.float32) + tot0
        pos0 = jnp.sum(c0 * o0, axis=0, keepdims=True)
        pos1 = jnp.sum(c1 * o1, axis=0, keepdims=True)
        pos0i = pos0.astype(jnp.int32)
        pos1i = pos1.astype(jnp.int32)
        e0b = e0[:, sl]
        e1b = e1[:, sl]

        g0 = e0b * (B * cap) + b * cap + jnp.minimum(pos0i, cap - 1)
        g1 = e1b * (B * cap) + b * cap + jnp.minimum(pos1i, cap - 1)
        g01_ref[0, sl] = g0[0]
        g01_ref[1, sl] = g1[0]
        s0 = e0b * (B * cap) + b * cap + pos0i
        s1 = e1b * (B * cap) + b * cap + pos1i
        s01_ref[0, sl] = jnp.where(pos0i < cap, s0, nrows)[0]
        s01_ref[1, sl] = jnp.where(pos1i < cap, s1, nrows)[0]
        p01_ref[0, :, sl] = jnp.broadcast_to(m0[:, sl], (16, T))
        p01_ref[1, :, sl] = jnp.broadcast_to(m1[:, sl], (16, T))


def _expert_body(xe_ref, w1_ref, w2_ref, w3_ref, eo_ref, oacc_ref, *, n_h):
    h = pl.program_id(1)
    xe = xe_ref[...].astype(jnp.bfloat16)
    hh = jnp.dot(xe, w1_ref[0].astype(jnp.bfloat16),
                 preferred_element_type=jnp.float32)
    gg = jnp.dot(xe, w2_ref[0].astype(jnp.bfloat16),
                 preferred_element_type=jnp.float32)
    gg = gg * (1.0 / (1.0 + jnp.exp(-gg)))
    act = gg * hh
    o_tile = jnp.dot(act.astype(jnp.bfloat16),
                     w3_ref[0].astype(jnp.bfloat16),
                     preferred_element_type=jnp.float32)

    @pl.when(h == 0)
    def _init():
        oacc_ref[...] = o_tile

    @pl.when(h != 0)
    def _acc():
        oacc_ref[...] = oacc_ref[...] + o_tile

    @pl.when(h == n_h - 1)
    def _out():
        eo_ref[...] = oacc_ref[...]


def _sc_scatter_body(x_hbm, s_hbm, out_hbm, idx0_v, idx1_v, rows0_v,
                     rows1_v, semr0, semr1, semw0, semw1, *,
                     asg_per_w, chunk, BT):
    wid = lax.axis_index("s") * NC + lax.axis_index("c")
    n = asg_per_w // chunk
    a_base = wid * asg_per_w
    bt_base = lax.rem(a_base, BT)
    idx = (idx0_v, idx1_v)
    rows = (rows0_v, rows1_v)
    semr = (semr0, semr1)
    semw = (semw0, semw1)
    reads = [None, None]
    wrb = [None, None]

    def fire(ch):
        p = ch % 2
        pltpu.sync_copy(s_hbm.at[pl.ds(a_base + ch * chunk, chunk)], idx[p])
        reads[p] = pltpu.async_copy(
            x_hbm.at[pl.ds(bt_base + ch * chunk, chunk)], rows[p], semr[p])

    fire(0)
    for ch in range(n):
        p = ch % 2
        if ch + 1 < n:
            if wrb[1 - p] is not None:
                wrb[1 - p].wait()
            fire(ch + 1)
        reads[p].wait()
        wrb[p] = pltpu.async_copy(rows[p], out_hbm.at[idx[p]], semw[p])
    for p in range(2):
        if wrb[p] is not None:
            wrb[p].wait()


def _sc_combine_body(eo_hbm, g0_hbm, g1_hbm, p0_hbm, p1_hbm, y_hbm,
                     g0a, g1a, p0a, p1a, r0a, r1a,
                     g0b, g1b, p0b, p1b, r0b, r1b,
                     sga0, sga1, sgb0, sgb1, swa, swb, *,
                     tok_per_w, chunk, C):
    wid = lax.axis_index("s") * NC + lax.axis_index("c")
    nv = C // NLANE
    n = tok_per_w // chunk
    bufs = ((g0a, g1a, p0a, p1a, r0a, r1a, sga0, sga1, swa),
            (g0b, g1b, p0b, p1b, r0b, r1b, sgb0, sgb1, swb))
    gath = [None, None]
    wrb = [None, None]

    def fire(ch):
        p = ch % 2
        G0, G1, P0, P1, R0, R1, SG0, SG1, _ = bufs[p]
        base = wid * tok_per_w + ch * chunk
        pltpu.sync_copy(g0_hbm.at[pl.ds(base, chunk)], G0)
        pltpu.sync_copy(g1_hbm.at[pl.ds(base, chunk)], G1)
        pltpu.sync_copy(p0_hbm.at[pl.ds(base, chunk)], P0)
        pltpu.sync_copy(p1_hbm.at[pl.ds(base, chunk)], P1)
        gath[p] = (pltpu.async_copy(eo_hbm.at[G0], R0, SG0),
                   pltpu.async_copy(eo_hbm.at[G1], R1, SG1))

    fire(0)
    for ch in range(n):
        p = ch % 2
        _, _, P0, P1, R0, R1, _, _, SW = bufs[p]
        if ch + 1 < n:
            if wrb[1 - p] is not None:
                wrb[1 - p].wait()
            fire(ch + 1)
        gath[p][0].wait()
        gath[p][1].wait()

        def row_body(rr, carry):
            a0 = P0[rr, :]
            a1 = P1[rr, :]
            for v in range(nv):
                x0 = R0[rr, pl.ds(v * NLANE, NLANE)]
                x1 = R1[rr, pl.ds(v * NLANE, NLANE)]
                R0[rr, pl.ds(v * NLANE, NLANE)] = a0 * x0 + a1 * x1
            return carry

        lax.fori_loop(0, chunk, row_body, 0)
        base = wid * tok_per_w + ch * chunk
        wrb[p] = pltpu.async_copy(R0, y_hbm.at[pl.ds(base, chunk)], SW)
    for p in range(2):
        if wrb[p] is not None:
            wrb[p].wait()


def kernel(x, W_r, w_c_fc, w_gate, w_c_proj):
    B, T, C = x.shape
    E, _, H = w_c_fc.shape
    cap = int(LOAD_FACTOR * TOP_K * max(1, T / E))
    NROWS = E * B * cap
    xf = x.reshape(B * T, C)

    p01, g01, s01 = pl.pallas_call(
        functools.partial(_route_body, B=B, T=T, cap=cap),
        out_shape=(
            jax.ShapeDtypeStruct((TOP_K, 16, B * T), jnp.float32),
            jax.ShapeDtypeStruct((TOP_K, B * T), jnp.int32),
            jax.ShapeDtypeStruct((TOP_K, B * T), jnp.int32),
        ),
    )(xf, W_r)
    s_flat = s01.reshape(TOP_K * B * T)
    p0t = p01[0].T
    p1t = p01[1].T

    mesh = plsc.VectorSubcoreMesh(core_axis_name="c", subcore_axis_name="s")
    asg_per_w = (TOP_K * B * T) // NW
    dchunk = 64
    xep = pl.kernel(
        functools.partial(_sc_scatter_body, asg_per_w=asg_per_w,
                          chunk=dchunk, BT=B * T),
        out_type=jax.ShapeDtypeStruct((NROWS + 8, C), jnp.float32),
        mesh=mesh,
        scratch_types=[
            pltpu.VMEM((dchunk,), jnp.int32),
            pltpu.VMEM((dchunk,), jnp.int32),
            pltpu.VMEM((dchunk, C), jnp.float32),
            pltpu.VMEM((dchunk, C), jnp.float32),
            pltpu.SemaphoreType.DMA,
            pltpu.SemaphoreType.DMA,
            pltpu.SemaphoreType.DMA,
            pltpu.SemaphoreType.DMA,
        ],
    )(xf, s_flat)

    HT = 768
    n_h = H // HT
    eo = pl.pallas_call(
        functools.partial(_expert_body, n_h=n_h),
        grid=(E, n_h),
        in_specs=[
            pl.BlockSpec((B * cap, C), lambda e, h: (e, 0)),
            pl.BlockSpec((1, C, HT), lambda e, h: (e, 0, h)),
            pl.BlockSpec((1, C, HT), lambda e, h: (e, 0, h)),
            pl.BlockSpec((1, HT, C), lambda e, h: (e, h, 0)),
        ],
        out_specs=pl.BlockSpec((B * cap, C), lambda e, h: (e, 0)),
        out_shape=jax.ShapeDtypeStruct((NROWS, C), jnp.float32),
        scratch_shapes=[pltpu.VMEM((B * cap, C), jnp.float32)],
        compiler_params=pltpu.CompilerParams(
            vmem_limit_bytes=100 * 1024 * 1024,
        ),
    )(xep, w_c_fc, w_gate, w_c_proj)

    tok_per_w = (B * T) // NW
    cchunk = 32
    y = pl.kernel(
        functools.partial(_sc_combine_body, tok_per_w=tok_per_w,
                          chunk=cchunk, C=C),
        out_type=jax.ShapeDtypeStruct((B * T, C), jnp.float32),
        mesh=mesh,
        scratch_types=(
            [pltpu.VMEM((cchunk,), jnp.int32)] * 2
            + [pltpu.VMEM((cchunk, 16), jnp.float32)] * 2
            + [pltpu.VMEM((cchunk, C), jnp.float32)] * 2
            + [pltpu.VMEM((cchunk,), jnp.int32)] * 2
            + [pltpu.VMEM((cchunk, 16), jnp.float32)] * 2
            + [pltpu.VMEM((cchunk, C), jnp.float32)] * 2
            + [pltpu.SemaphoreType.DMA] * 6
        ),
    )(eo, g01[0], g01[1], p0t, p1t)

    return y.reshape(B, T, C)
